# baseline (speedup 1.0000x reference)
import functools

import jax
import jax.numpy as jnp
from jax.experimental import pallas as pl

_ROWS_PER_BLOCK = 256
_LOG2E = 1.4426950408889634


def _sigmoid_of_relu(x):
    e = jax.lax.exp2(x * (-_LOG2E))
    return jnp.maximum(1.0 / (1.0 + e), 0.5)


def _fused_body(z_ref, w_ref, o_ref, *, k: int):
    x = jax.lax.dot_general(
        z_ref[...],
        w_ref[...],
        (((1,), (1,)), ((), ())),
        preferred_element_type=jnp.float32,
    )

    if k <= 0:
        o_ref[...] = _sigmoid_of_relu(x)
        return

    pos = jnp.where(x > 0.0, 1.0, 0.0)
    ones = jnp.ones((x.shape[1], 1), jnp.float32)
    n_pos = jax.lax.dot_general(
        pos, ones, (((1,), (0,)), ((), ())),
        preferred_element_type=jnp.float32,
    )
    need_search = jnp.any(n_pos > float(k))

    @pl.when(jnp.logical_not(need_search))
    def _common():
        o_ref[...] = _sigmoid_of_relu(x)

    @pl.when(need_search)
    def _rare():
        f = jnp.maximum(x, 0.0)
        bits = jax.lax.bitcast_convert_type(f, jnp.int32)

        def step(_, carry):
            lo, hi = carry
            mid = lo + ((hi - lo + 1) >> 1)
            cnt = jnp.sum((bits >= mid).astype(jnp.int32), axis=1,
                          keepdims=True)
            pred = cnt >= k
            lo = jnp.where(pred, mid, lo)
            hi = jnp.where(pred, hi, mid - 1)
            return lo, hi

        rows, cols = f.shape
        lo0 = jnp.zeros((rows, 1), jnp.int32)
        hi0 = jnp.full((rows, 1), jnp.int32(0x7F7FFFFF))
        lo, _ = jax.lax.fori_loop(0, 31, step, (lo0, hi0))

        gt = bits > lo
        ties = bits == lo
        m = k - jnp.sum(gt.astype(jnp.int32), axis=1, keepdims=True)
        col = jax.lax.broadcasted_iota(jnp.int32, f.shape, 1)

        def col_step(_, carry):
            clo, chi = carry
            cmid = clo + ((chi - clo + 1) >> 1)
            cnt = jnp.sum((ties & (col < cmid)).astype(jnp.int32), axis=1,
                          keepdims=True)
            pred = cnt <= m
            clo = jnp.where(pred, cmid, clo)
            chi = jnp.where(pred, chi, cmid - 1)
            return clo, chi

        nbits = max(1, (cols + 1).bit_length())
        clo0 = jnp.zeros((rows, 1), jnp.int32)
        chi0 = jnp.full((rows, 1), jnp.int32(cols))
        clo, _ = jax.lax.fori_loop(0, nbits, col_step, (clo0, chi0))
        mask = gt | (ties & (col < clo))
        o_ref[...] = jax.nn.sigmoid(jnp.where(mask, f, 0.0))


@jax.jit
def kernel(z, W, thresholds):
    n_rows, d = z.shape
    n_factors = W.shape[0]
    target_sparsity = 0.3
    k = int(n_factors * (1.0 - target_sparsity))

    z_aug = jnp.concatenate(
        [z.astype(jnp.float32), jnp.ones((n_rows, 1), jnp.float32)], axis=1)
    w_aug = jnp.concatenate(
        [W.astype(jnp.float32),
         -thresholds.reshape(n_factors, 1).astype(jnp.float32)], axis=1)
    d_aug = d + 1

    rb = min(_ROWS_PER_BLOCK, n_rows)
    grid = (n_rows // rb,)

    out = pl.pallas_call(
        functools.partial(_fused_body, k=k),
        grid=grid,
        in_specs=[
            pl.BlockSpec((rb, d_aug), lambda i: (i, 0)),
            pl.BlockSpec((n_factors, d_aug), lambda i: (0, 0)),
        ],
        out_specs=pl.BlockSpec((rb, n_factors), lambda i: (i, 0)),
        out_shape=jax.ShapeDtypeStruct((n_rows, n_factors), jnp.float32),
    )(z_aug, w_aug)
    return out


# speedup vs baseline: 1.1471x; 1.1471x over previous
import functools

import jax
import jax.numpy as jnp
from jax.experimental import pallas as pl

_ROWS_PER_BLOCK = 256
_LOG2E = 1.4426950408889634


def _sigmoid_of_relu_from_h(h):
    e = jax.lax.exp2(h * _LOG2E)
    return jnp.maximum(1.0 / (1.0 + e), 0.5)


def _fused_body(z_ref, w_ref, t_ref, o_ref, *, k: int):
    mm = jax.lax.dot_general(
        z_ref[...],
        w_ref[...],
        (((1,), (1,)), ((), ())),
        preferred_element_type=jnp.float32,
    )
    h = t_ref[...] - mm

    if k <= 0:
        o_ref[...] = _sigmoid_of_relu_from_h(h)
        return

    pos = jnp.where(h < 0.0, 1.0, 0.0)
    ones = jnp.ones((h.shape[1], 1), jnp.float32)
    n_pos = jax.lax.dot_general(
        pos, ones, (((1,), (0,)), ((), ())),
        preferred_element_type=jnp.float32,
    )
    need_search = jnp.any(n_pos > float(k))

    @pl.when(jnp.logical_not(need_search))
    def _common():
        o_ref[...] = _sigmoid_of_relu_from_h(h)

    @pl.when(need_search)
    def _rare():
        f = jnp.maximum(-h, 0.0)
        bits = jax.lax.bitcast_convert_type(f, jnp.int32)

        def step(_, carry):
            lo, hi = carry
            mid = lo + ((hi - lo + 1) >> 1)
            cnt = jnp.sum((bits >= mid).astype(jnp.int32), axis=1,
                          keepdims=True)
            pred = cnt >= k
            lo = jnp.where(pred, mid, lo)
            hi = jnp.where(pred, hi, mid - 1)
            return lo, hi

        rows, cols = f.shape
        lo0 = jnp.zeros((rows, 1), jnp.int32)
        hi0 = jnp.full((rows, 1), jnp.int32(0x7F7FFFFF))
        lo, _ = jax.lax.fori_loop(0, 31, step, (lo0, hi0))

        gt = bits > lo
        ties = bits == lo
        m = k - jnp.sum(gt.astype(jnp.int32), axis=1, keepdims=True)
        col = jax.lax.broadcasted_iota(jnp.int32, f.shape, 1)

        def col_step(_, carry):
            clo, chi = carry
            cmid = clo + ((chi - clo + 1) >> 1)
            cnt = jnp.sum((ties & (col < cmid)).astype(jnp.int32), axis=1,
                          keepdims=True)
            pred = cnt <= m
            clo = jnp.where(pred, cmid, clo)
            chi = jnp.where(pred, chi, cmid - 1)
            return clo, chi

        nbits = max(1, (cols + 1).bit_length())
        clo0 = jnp.zeros((rows, 1), jnp.int32)
        chi0 = jnp.full((rows, 1), jnp.int32(cols))
        clo, _ = jax.lax.fori_loop(0, nbits, col_step, (clo0, chi0))
        mask = gt | (ties & (col < clo))
        o_ref[...] = jax.nn.sigmoid(jnp.where(mask, f, 0.0))


@jax.jit
def kernel(z, W, thresholds):
    n_rows, d = z.shape
    n_factors = W.shape[0]
    target_sparsity = 0.3
    k = int(n_factors * (1.0 - target_sparsity))

    th2d = thresholds.reshape(1, n_factors).astype(jnp.float32)
    rb = min(_ROWS_PER_BLOCK, n_rows)
    grid = (n_rows // rb,)

    out = pl.pallas_call(
        functools.partial(_fused_body, k=k),
        grid=grid,
        in_specs=[
            pl.BlockSpec((rb, d), lambda i: (i, 0)),
            pl.BlockSpec((n_factors, d), lambda i: (0, 0)),
            pl.BlockSpec((1, n_factors), lambda i: (0, 0)),
        ],
        out_specs=pl.BlockSpec((rb, n_factors), lambda i: (i, 0)),
        out_shape=jax.ShapeDtypeStruct((n_rows, n_factors), jnp.float32),
    )(z.astype(jnp.float32), W.astype(jnp.float32), th2d)
    return out


# speedup vs baseline: 1.2910x; 1.1254x over previous
import functools

import jax
import jax.numpy as jnp
from jax.experimental import pallas as pl

_ROWS_PER_BLOCK = 256
_LOG2E = 1.4426950408889634


def _sigmoid_of_relu_from_h(h):
    e = jax.lax.exp2(h * _LOG2E)
    return jnp.maximum(1.0 / (1.0 + e), 0.5)


def _fused_body(z_ref, w_ref, t_ref, o_ref, *, k: int):
    mm = jax.lax.dot_general(
        z_ref[...],
        w_ref[...],
        (((1,), (1,)), ((), ())),
        preferred_element_type=jnp.float32,
    )
    h = t_ref[...] - mm

    o_ref[...] = _sigmoid_of_relu_from_h(h)

    if k <= 0:
        return

    pos = jnp.where(h < 0.0, 1.0, 0.0).astype(jnp.bfloat16)
    ones = jnp.ones((h.shape[1], 1), jnp.bfloat16)
    n_pos = jax.lax.dot_general(
        pos, ones, (((1,), (0,)), ((), ())),
        preferred_element_type=jnp.float32,
    )
    need_search = jnp.any(n_pos > float(k))

    @pl.when(need_search)
    def _rare():
        f = jnp.maximum(-h, 0.0)
        bits = jax.lax.bitcast_convert_type(f, jnp.int32)

        def step(_, carry):
            lo, hi = carry
            mid = lo + ((hi - lo + 1) >> 1)
            cnt = jnp.sum((bits >= mid).astype(jnp.int32), axis=1,
                          keepdims=True)
            pred = cnt >= k
            lo = jnp.where(pred, mid, lo)
            hi = jnp.where(pred, hi, mid - 1)
            return lo, hi

        rows, cols = f.shape
        lo0 = jnp.zeros((rows, 1), jnp.int32)
        hi0 = jnp.full((rows, 1), jnp.int32(0x7F7FFFFF))
        lo, _ = jax.lax.fori_loop(0, 31, step, (lo0, hi0))

        gt = bits > lo
        ties = bits == lo
        m = k - jnp.sum(gt.astype(jnp.int32), axis=1, keepdims=True)
        col = jax.lax.broadcasted_iota(jnp.int32, f.shape, 1)

        def col_step(_, carry):
            clo, chi = carry
            cmid = clo + ((chi - clo + 1) >> 1)
            cnt = jnp.sum((ties & (col < cmid)).astype(jnp.int32), axis=1,
                          keepdims=True)
            pred = cnt <= m
            clo = jnp.where(pred, cmid, clo)
            chi = jnp.where(pred, chi, cmid - 1)
            return clo, chi

        nbits = max(1, (cols + 1).bit_length())
        clo0 = jnp.zeros((rows, 1), jnp.int32)
        chi0 = jnp.full((rows, 1), jnp.int32(cols))
        clo, _ = jax.lax.fori_loop(0, nbits, col_step, (clo0, chi0))
        mask = gt | (ties & (col < clo))
        o_ref[...] = jax.nn.sigmoid(jnp.where(mask, f, 0.0))


@jax.jit
def kernel(z, W, thresholds):
    n_rows, d = z.shape
    n_factors = W.shape[0]
    target_sparsity = 0.3
    k = int(n_factors * (1.0 - target_sparsity))

    th2d = thresholds.reshape(1, n_factors).astype(jnp.float32)
    rb = min(_ROWS_PER_BLOCK, n_rows)
    grid = (n_rows // rb,)

    out = pl.pallas_call(
        functools.partial(_fused_body, k=k),
        grid=grid,
        in_specs=[
            pl.BlockSpec((rb, d), lambda i: (i, 0)),
            pl.BlockSpec((n_factors, d), lambda i: (0, 0)),
            pl.BlockSpec((1, n_factors), lambda i: (0, 0)),
        ],
        out_specs=pl.BlockSpec((rb, n_factors), lambda i: (i, 0)),
        out_shape=jax.ShapeDtypeStruct((n_rows, n_factors), jnp.float32),
    )(z.astype(jnp.float32), W.astype(jnp.float32), th2d)
    return out
